# baseline (speedup 1.0000x reference)
import functools

import jax
import jax.numpy as jnp
from jax import lax
from jax.experimental import pallas as pl
from jax.experimental.pallas import tpu as pltpu
from jax.experimental.pallas import tpu_sc as plsc

NC = 2
NS = 16
NW = NC * NS
LANES = 16
K = 16
UNROLL = 4


def _sigmoid(x):
    return 1.0 / (1.0 + jnp.exp(-x))


def _softplus(x):
    u = jnp.exp(-jnp.abs(x))
    t = u / (u + 2.0)
    w = t * t
    p = 1.0 + w * (1.0 / 3.0 + w * (0.2 + w * (1.0 / 7.0 + w * (1.0 / 9.0))))
    return jnp.maximum(x, 0.0) + 2.0 * t * p


def _make_edge_kernel(NPAD, nb, HH):
    EPT = nb * K
    RW = HH + LANES
    RPT = NPAD // NS
    W2H = 2 * HH
    mesh = plsc.VectorSubcoreMesh(core_axis_name="c", subcore_axis_name="s")

    @functools.partial(
        pl.kernel,
        out_type=jax.ShapeDtypeStruct((NC, NPAD, RW), jnp.float32),
        mesh=mesh,
        compiler_params=pltpu.CompilerParams(use_tc_tiling_on_sc=False),
        scratch_types=[
            pltpu.VMEM((K, W2H), jnp.float32),
            pltpu.VMEM((K, W2H), jnp.float32),
            pltpu.VMEM((K, W2H), jnp.float32),
            pltpu.VMEM((K, W2H), jnp.float32),
            pltpu.VMEM((K, W2H), jnp.float32),
            pltpu.VMEM((K, W2H), jnp.float32),
            pltpu.VMEM((K, RW), jnp.float32),
            pltpu.VMEM((K, RW), jnp.float32),
            pltpu.VMEM((K,), jnp.int32),
            pltpu.VMEM((K,), jnp.int32),
            pltpu.VMEM((K,), jnp.int32),
            pltpu.VMEM((K,), jnp.int32),
            pltpu.VMEM((K,), jnp.int32),
            pltpu.VMEM((K,), jnp.int32),
            pltpu.VMEM((K,), jnp.int32),
            pltpu.VMEM((K,), jnp.int32),
            pltpu.VMEM_SHARED((NPAD, RW), jnp.float32),
            pltpu.SemaphoreType.DMA,
            pltpu.SemaphoreType.DMA,
            pltpu.SemaphoreType.DMA,
        ],
    )
    def edge_kernel(a_hbm, b_hbm, c_hbm, dst_hbm, src_hbm, out_hbm,
                    a0, a1, b0, b1, c0, c1, m0, m1,
                    d0, d1, d2, d3, s0, s1, s2, s3,
                    shared, sem_g0, sem_g1, sem_s):
        cid = lax.axis_index("c")
        sid = lax.axis_index("s")
        w = cid * NS + sid
        abufs = (a0, a1)
        bbufs = (b0, b1)
        cbufs = (c0, c1)
        mbufs = (m0, m1)
        drefs = (d0, d1, d2, d3)
        srefs = (s0, s1, s2, s3)
        gsems = (sem_g0, sem_g1)

        zvec = jnp.zeros((LANES,), jnp.float32)

        @pl.loop(0, K)
        def _zero_m(j):
            for qq in range(RW // LANES):
                m0[j, pl.ds(qq * LANES, LANES)] = zvec

        base = sid * RPT
        nfull = RPT // K
        rem = RPT - nfull * K

        @pl.loop(0, nfull)
        def _zcp(z):
            pltpu.sync_copy(m0, shared.at[pl.ds(base + z * K, K)])

        if rem:
            pltpu.sync_copy(m0.at[pl.ds(0, rem)],
                            shared.at[pl.ds(base + nfull * K, rem)])

        ji = lax.iota(jnp.int32, LANES)
        unit = jnp.where(ji == 0, 1.0, 0.0).astype(jnp.float32)

        @pl.loop(0, K)
        def _pad_m(j):
            m0[j, pl.ds(HH, LANES)] = unit
            m1[j, pl.ds(HH, LANES)] = unit

        plsc.subcore_barrier()

        pltpu.sync_copy(dst_hbm.at[w, 0], d0)
        pltpu.sync_copy(src_hbm.at[w, 0], s0)
        pltpu.async_copy(a_hbm.at[d0], a0, sem_g0)
        pltpu.async_copy(b_hbm.at[s0], b0, sem_g0)
        pltpu.async_copy(c_hbm.at[pl.ds(w * EPT, K)], c0, sem_g0)

        @pl.loop(0, nb, step=UNROLL)
        def _outer(B0):
            for j in range(UNROLL):
                b = B0 + j
                p = j & 1
                q = j & 3
                qn = (j + 1) & 3
                qp = (j + 3) & 3
                A_, B_, C_, M_ = abufs[p], bbufs[p], cbufs[p], mbufs[p]
                An, Bn, Cn = abufs[1 - p], bbufs[1 - p], cbufs[1 - p]
                Mp = mbufs[1 - p]
                pltpu.make_async_copy(a_hbm.at[drefs[q]], A_, gsems[p]).wait()
                pltpu.make_async_copy(b_hbm.at[srefs[q]], B_, gsems[p]).wait()
                pltpu.make_async_copy(
                    c_hbm.at[pl.ds(w * EPT + b * K, K)], C_, gsems[p]).wait()

                @pl.when(b + 1 < nb)
                def _pf():
                    pltpu.sync_copy(dst_hbm.at[w, b + 1], drefs[qn])
                    pltpu.sync_copy(src_hbm.at[w, b + 1], srefs[qn])
                    pltpu.async_copy(a_hbm.at[drefs[qn]], An, gsems[1 - p])
                    pltpu.async_copy(b_hbm.at[srefs[qn]], Bn, gsems[1 - p])
                    pltpu.async_copy(
                        c_hbm.at[pl.ds(w * EPT + (b + 1) * K, K)],
                        Cn, gsems[1 - p])

                @pl.loop(0, K)
                def _edge(jj):
                    for g in range(HH // LANES):
                        lo = g * LANES
                        zf = (A_[jj, pl.ds(lo, LANES)]
                              + B_[jj, pl.ds(lo, LANES)]
                              + C_[jj, pl.ds(lo, LANES)])
                        zs = (A_[jj, pl.ds(HH + lo, LANES)]
                              + B_[jj, pl.ds(HH + lo, LANES)]
                              + C_[jj, pl.ds(HH + lo, LANES)])
                        M_[jj, pl.ds(lo, LANES)] = (
                            _sigmoid(zf) * _softplus(zs))

                @pl.when(b > 0)
                def _ws():
                    pltpu.make_async_copy(
                        Mp, shared.at[drefs[qp]], sem_s).wait()

                pltpu.async_copy(M_, shared.at[drefs[q]], sem_s, add=True)

        pltpu.make_async_copy(m1, shared.at[d3], sem_s).wait()
        plsc.subcore_barrier()
        pltpu.sync_copy(shared.at[pl.ds(base, RPT)],
                        out_hbm.at[cid, pl.ds(base, RPT)])

    return edge_kernel


def kernel(x, edge_index, edge_attr, batch, emb, Wf, bf, Ws, bs,
           gamma, beta, W1, b1, W2, b2):
    N = x.shape[0]
    E = edge_index.shape[1]
    H = emb.shape[1]
    DE = edge_attr.shape[1]
    L = Wf.shape[0]
    FC = W1.shape[1]
    G = 64
    f32 = jnp.float32

    NPAD = ((N + LANES + 127) // 128) * 128
    nb = -(-E // (NW * K))
    nb = ((nb + UNROLL - 1) // UNROLL) * UNROLL
    EW = NW * nb * K
    RW = H + LANES

    idt = edge_index.dtype
    dstp = jnp.concatenate(
        [edge_index[1], jnp.full((EW - E,), N, idt)]).reshape(NW, nb, K)
    srcp = jnp.concatenate(
        [edge_index[0], jnp.full((EW - E,), N, idt)]).reshape(NW, nb, K)
    eap = jnp.concatenate(
        [edge_attr, jnp.zeros((EW - E, DE), f32)], axis=0)
    xp = jnp.concatenate([x, jnp.zeros((NPAD - N, 1), x.dtype)], axis=0)
    batchp = jnp.concatenate(
        [batch, jnp.full((NPAD - N,), G, batch.dtype)]).reshape(NPAD, 1)
    Wd = jnp.concatenate([Wf[:, :H, :], Ws[:, :H, :]], axis=2)
    Wsr = jnp.concatenate([Wf[:, H:2 * H, :], Ws[:, H:2 * H, :]], axis=2)
    We = jnp.concatenate([Wf[:, 2 * H:, :], Ws[:, 2 * H:, :]], axis=2)
    bcat = jnp.concatenate([bf, bs], axis=1)

    def _init_body(x_ref, emb_ref, wd_ref, ws_ref, h_ref, a_ref, b_ref):
        z = jnp.clip(x_ref[...], 0, 119)
        cols = lax.broadcasted_iota(jnp.int32, (NPAD, 120), 1)
        rows = lax.broadcasted_iota(jnp.int32, (NPAD, 1), 0)
        oh = jnp.where((z == cols) & (rows < N), 1.0, 0.0).astype(f32)
        h = jnp.dot(oh, emb_ref[...], preferred_element_type=f32)
        h_ref[...] = h
        a_ref[...] = jnp.dot(h, wd_ref[...], preferred_element_type=f32)
        b_ref[...] = jnp.dot(h, ws_ref[...], preferred_element_type=f32)

    def _c_body(ea_ref, w_ref, bv_ref, c_ref):
        c_ref[...] = (jnp.dot(ea_ref[...], w_ref[...],
                              preferred_element_type=f32) + bv_ref[...])

    def _node_core(parts_ref, h_ref, g_ref, be_ref):
        sm = parts_ref[0] + parts_ref[1]
        deg = jnp.maximum(sm[:, H:H + 1], 1.0)
        aggr = sm[:, :H] / deg
        rows = lax.broadcasted_iota(jnp.int32, (NPAD, 1), 0)
        msk = (rows < N).astype(f32)
        aggr = aggr * msk
        mu = jnp.sum(aggr, axis=0, keepdims=True) / N
        d = (aggr - mu) * msk
        var = jnp.sum(d * d, axis=0, keepdims=True) / N
        nrm = d * lax.rsqrt(var + 1e-5) * g_ref[...] + be_ref[...]
        return jnp.maximum(h_ref[...] + nrm, 0.0) * msk

    def _node_ab_body(parts_ref, h_ref, g_ref, be_ref, wd_ref, ws_ref,
                      h2_ref, a_ref, b_ref):
        h2 = _node_core(parts_ref, h_ref, g_ref, be_ref)
        h2_ref[...] = h2
        a_ref[...] = jnp.dot(h2, wd_ref[...], preferred_element_type=f32)
        b_ref[...] = jnp.dot(h2, ws_ref[...], preferred_element_type=f32)

    def _node_last_body(parts_ref, h_ref, g_ref, be_ref, h2_ref):
        h2_ref[...] = _node_core(parts_ref, h_ref, g_ref, be_ref)

    def _pool_body(h_ref, bt_ref, w1_ref, b1_ref, w2_ref, b2_ref, out_ref):
        cols = lax.broadcasted_iota(jnp.int32, (NPAD, G), 1)
        oh = (bt_ref[...] == cols).astype(f32)
        ones = jnp.ones((NPAD, 1), f32)
        cnt = jnp.maximum(
            lax.dot_general(oh, ones, (((0,), (0,)), ((), ())),
                            preferred_element_type=f32), 1.0)
        g = lax.dot_general(oh, h_ref[...], (((0,), (0,)), ((), ())),
                            preferred_element_type=f32)
        g = g / cnt
        y = jnp.maximum(jnp.dot(g, w1_ref[...], preferred_element_type=f32)
                        + b1_ref[...], 0.0)
        out_ref[...] = (jnp.dot(y, w2_ref[...], preferred_element_type=f32)
                        + b2_ref[...])

    sds = jax.ShapeDtypeStruct
    init_call = pl.pallas_call(
        _init_body,
        out_shape=(sds((NPAD, H), f32), sds((NPAD, 2 * H), f32),
                   sds((NPAD, 2 * H), f32)))

    RB = 2048
    assert EW % RB == 0
    c_call = pl.pallas_call(
        _c_body,
        grid=(EW // RB,),
        in_specs=[pl.BlockSpec((RB, DE), lambda i: (i, 0)),
                  pl.BlockSpec((DE, 2 * H), lambda i: (0, 0)),
                  pl.BlockSpec((1, 2 * H), lambda i: (0, 0))],
        out_specs=pl.BlockSpec((RB, 2 * H), lambda i: (i, 0)),
        out_shape=sds((EW, 2 * H), f32))

    node_ab_call = pl.pallas_call(
        _node_ab_body,
        out_shape=(sds((NPAD, H), f32), sds((NPAD, 2 * H), f32),
                   sds((NPAD, 2 * H), f32)))
    node_last_call = pl.pallas_call(_node_last_body,
                                    out_shape=sds((NPAD, H), f32))
    pool_call = pl.pallas_call(_pool_body, out_shape=sds((G, 1), f32))

    edge_call = _make_edge_kernel(NPAD, nb, H)

    h, A, B = init_call(xp, emb, Wd[0], Wsr[0])
    for l in range(L):
        Cl = c_call(eap, We[l], bcat[l].reshape(1, -1))
        parts = edge_call(A, B, Cl, dstp, srcp)
        if l < L - 1:
            h, A, B = node_ab_call(parts, h, gamma[l].reshape(1, -1),
                                   beta[l].reshape(1, -1), Wd[l + 1],
                                   Wsr[l + 1])
        else:
            h = node_last_call(parts, h, gamma[l].reshape(1, -1),
                               beta[l].reshape(1, -1))
    out = pool_call(h, batchp, W1, b1.reshape(1, -1), W2, b2.reshape(1, 1))
    return out.reshape(-1)


# speedup vs baseline: 1.1062x; 1.1062x over previous
import functools

import jax
import jax.numpy as jnp
from jax import lax
from jax.experimental import pallas as pl
from jax.experimental.pallas import tpu as pltpu
from jax.experimental.pallas import tpu_sc as plsc

NC = 2
NS = 16
NW = NC * NS
LANES = 16
K = 16
CH = 64


def _sigmoid(x):
    return 1.0 / (1.0 + jnp.exp(-x))


def _softplus(x):
    u = jnp.exp(-jnp.abs(x))
    t = u / (u + 2.0)
    w = t * t
    p = 1.0 + w * (1.0 / 3.0 + w * (0.2 + w * (1.0 / 7.0 + w * (1.0 / 9.0))))
    return jnp.maximum(x, 0.0) + 2.0 * t * p


def _make_edge_kernel(NPAD, nb, HH):
    EPT = nb * K
    RW = HH + LANES
    RPT = NPAD // NS
    W2H = 2 * HH
    NCHUNK = nb // CH
    mesh = plsc.VectorSubcoreMesh(core_axis_name="c", subcore_axis_name="s")

    @functools.partial(
        pl.kernel,
        out_type=jax.ShapeDtypeStruct((NC, NPAD, RW), jnp.float32),
        mesh=mesh,
        compiler_params=pltpu.CompilerParams(use_tc_tiling_on_sc=False),
        scratch_types=[
            pltpu.VMEM((K, W2H), jnp.float32),
            pltpu.VMEM((K, W2H), jnp.float32),
            pltpu.VMEM((K, W2H), jnp.float32),
            pltpu.VMEM((K, W2H), jnp.float32),
            pltpu.VMEM((K, W2H), jnp.float32),
            pltpu.VMEM((K, W2H), jnp.float32),
            pltpu.VMEM((K, RW), jnp.float32),
            pltpu.VMEM((K, RW), jnp.float32),
            pltpu.VMEM((CH, K), jnp.int32),
            pltpu.VMEM((CH, K), jnp.int32),
            pltpu.VMEM((K,), jnp.int32),
            pltpu.VMEM((K,), jnp.int32),
            pltpu.VMEM_SHARED((NPAD, RW), jnp.float32),
            pltpu.SemaphoreType.DMA,
            pltpu.SemaphoreType.DMA,
            pltpu.SemaphoreType.DMA,
        ],
    )
    def edge_kernel(a_hbm, b_hbm, c_hbm, dst_hbm, src_hbm, out_hbm,
                    a0, a1, b0, b1, c0, c1, m0, m1, dch, sch, dsc0, dsc1,
                    shared, sem_g0, sem_g1, sem_s):
        cid = lax.axis_index("c")
        sid = lax.axis_index("s")
        w = cid * NS + sid
        abufs = (a0, a1)
        bbufs = (b0, b1)
        cbufs = (c0, c1)
        mbufs = (m0, m1)
        dscs = (dsc0, dsc1)
        gsems = (sem_g0, sem_g1)

        zvec = jnp.zeros((LANES,), jnp.float32)

        @pl.loop(0, K)
        def _zero_m(j):
            for qq in range(RW // LANES):
                m0[j, pl.ds(qq * LANES, LANES)] = zvec

        base = sid * RPT
        nfull = RPT // K
        rem = RPT - nfull * K

        @pl.loop(0, nfull)
        def _zcp(z):
            pltpu.sync_copy(m0, shared.at[pl.ds(base + z * K, K)])

        if rem:
            pltpu.sync_copy(m0.at[pl.ds(0, rem)],
                            shared.at[pl.ds(base + nfull * K, rem)])

        ji = lax.iota(jnp.int32, LANES)
        unit = jnp.where(ji == 0, 1.0, 0.0).astype(jnp.float32)

        @pl.loop(0, K)
        def _pad_m(j):
            m0[j, pl.ds(HH, LANES)] = unit
            m1[j, pl.ds(HH, LANES)] = unit

        plsc.subcore_barrier()

        @pl.loop(0, NCHUNK)
        def _chunk(c):
            pltpu.sync_copy(dst_hbm.at[w, c], dch)
            pltpu.sync_copy(src_hbm.at[w, c], sch)
            base_e = w * EPT + c * (CH * K)
            pltpu.async_copy(a_hbm.at[dch.at[0]], a0, sem_g0)
            pltpu.async_copy(b_hbm.at[sch.at[0]], b0, sem_g0)
            pltpu.async_copy(c_hbm.at[pl.ds(base_e, K)], c0, sem_g0)

            @pl.loop(0, CH, step=2)
            def _blk(J):
                for j in range(2):
                    p = j
                    b = J + j
                    A_, B_, C_, M_ = abufs[p], bbufs[p], cbufs[p], mbufs[p]
                    An, Bn, Cn = abufs[1 - p], bbufs[1 - p], cbufs[1 - p]
                    Mp = mbufs[1 - p]
                    pltpu.make_async_copy(
                        a_hbm.at[dch.at[b]], A_, gsems[p]).wait()
                    pltpu.make_async_copy(
                        b_hbm.at[sch.at[b]], B_, gsems[p]).wait()
                    pltpu.make_async_copy(
                        c_hbm.at[pl.ds(base_e + b * K, K)], C_,
                        gsems[p]).wait()

                    @pl.when(b + 1 < CH)
                    def _pf():
                        pltpu.async_copy(
                            a_hbm.at[dch.at[b + 1]], An, gsems[1 - p])
                        pltpu.async_copy(
                            b_hbm.at[sch.at[b + 1]], Bn, gsems[1 - p])
                        pltpu.async_copy(
                            c_hbm.at[pl.ds(base_e + (b + 1) * K, K)],
                            Cn, gsems[1 - p])

                    @pl.loop(0, K)
                    def _edge(jj):
                        for g in range(HH // LANES):
                            lo = g * LANES
                            zf = (A_[jj, pl.ds(lo, LANES)]
                                  + B_[jj, pl.ds(lo, LANES)]
                                  + C_[jj, pl.ds(lo, LANES)])
                            zs = (A_[jj, pl.ds(HH + lo, LANES)]
                                  + B_[jj, pl.ds(HH + lo, LANES)]
                                  + C_[jj, pl.ds(HH + lo, LANES)])
                            M_[jj, pl.ds(lo, LANES)] = (
                                _sigmoid(zf) * _softplus(zs))

                    dscs[p][...] = dch[b]

                    @pl.when(b > 0)
                    def _ws():
                        pltpu.make_async_copy(
                            Mp, shared.at[dscs[1 - p]], sem_s).wait()

                    pltpu.async_copy(M_, shared.at[dscs[p]], sem_s,
                                     add=True)

            pltpu.make_async_copy(m1, shared.at[dsc1], sem_s).wait()

        plsc.subcore_barrier()
        pltpu.sync_copy(shared.at[pl.ds(base, RPT)],
                        out_hbm.at[cid, pl.ds(base, RPT)])

    return edge_kernel


def kernel(x, edge_index, edge_attr, batch, emb, Wf, bf, Ws, bs,
           gamma, beta, W1, b1, W2, b2):
    N = x.shape[0]
    E = edge_index.shape[1]
    H = emb.shape[1]
    DE = edge_attr.shape[1]
    L = Wf.shape[0]
    FC = W1.shape[1]
    G = 64
    f32 = jnp.float32

    NPAD = ((N + LANES + 127) // 128) * 128
    nb = -(-E // (NW * K))
    nb = ((nb + CH - 1) // CH) * CH
    EW = NW * nb * K
    RW = H + LANES

    idt = edge_index.dtype
    dstp = jnp.concatenate(
        [edge_index[1], jnp.full((EW - E,), N, idt)]).reshape(
            NW, nb // CH, CH, K)
    srcp = jnp.concatenate(
        [edge_index[0], jnp.full((EW - E,), N, idt)]).reshape(
            NW, nb // CH, CH, K)
    eap = jnp.concatenate(
        [edge_attr, jnp.zeros((EW - E, DE), f32)], axis=0)
    xp = jnp.concatenate([x, jnp.zeros((NPAD - N, 1), x.dtype)], axis=0)
    batchp = jnp.concatenate(
        [batch, jnp.full((NPAD - N,), G, batch.dtype)]).reshape(NPAD, 1)
    Wd = jnp.concatenate([Wf[:, :H, :], Ws[:, :H, :]], axis=2)
    Wsr = jnp.concatenate([Wf[:, H:2 * H, :], Ws[:, H:2 * H, :]], axis=2)
    We = jnp.concatenate([Wf[:, 2 * H:, :], Ws[:, 2 * H:, :]], axis=2)
    bcat = jnp.concatenate([bf, bs], axis=1)

    def _init_body(x_ref, emb_ref, wd_ref, ws_ref, h_ref, a_ref, b_ref):
        z = jnp.clip(x_ref[...], 0, 119)
        cols = lax.broadcasted_iota(jnp.int32, (NPAD, 120), 1)
        rows = lax.broadcasted_iota(jnp.int32, (NPAD, 1), 0)
        oh = jnp.where((z == cols) & (rows < N), 1.0, 0.0).astype(f32)
        h = jnp.dot(oh, emb_ref[...], preferred_element_type=f32)
        h_ref[...] = h
        a_ref[...] = jnp.dot(h, wd_ref[...], preferred_element_type=f32)
        b_ref[...] = jnp.dot(h, ws_ref[...], preferred_element_type=f32)

    def _c_body(ea_ref, w_ref, bv_ref, c_ref):
        c_ref[...] = (jnp.dot(ea_ref[...], w_ref[...],
                              preferred_element_type=f32) + bv_ref[...])

    def _node_core(parts_ref, h_ref, g_ref, be_ref):
        sm = parts_ref[0] + parts_ref[1]
        deg = jnp.maximum(sm[:, H:H + 1], 1.0)
        aggr = sm[:, :H] / deg
        rows = lax.broadcasted_iota(jnp.int32, (NPAD, 1), 0)
        msk = (rows < N).astype(f32)
        aggr = aggr * msk
        mu = jnp.sum(aggr, axis=0, keepdims=True) / N
        d = (aggr - mu) * msk
        var = jnp.sum(d * d, axis=0, keepdims=True) / N
        nrm = d * lax.rsqrt(var + 1e-5) * g_ref[...] + be_ref[...]
        return jnp.maximum(h_ref[...] + nrm, 0.0) * msk

    def _node_ab_body(parts_ref, h_ref, g_ref, be_ref, wd_ref, ws_ref,
                      h2_ref, a_ref, b_ref):
        h2 = _node_core(parts_ref, h_ref, g_ref, be_ref)
        h2_ref[...] = h2
        a_ref[...] = jnp.dot(h2, wd_ref[...], preferred_element_type=f32)
        b_ref[...] = jnp.dot(h2, ws_ref[...], preferred_element_type=f32)

    def _node_last_body(parts_ref, h_ref, g_ref, be_ref, h2_ref):
        h2_ref[...] = _node_core(parts_ref, h_ref, g_ref, be_ref)

    def _pool_body(h_ref, bt_ref, w1_ref, b1_ref, w2_ref, b2_ref, out_ref):
        cols = lax.broadcasted_iota(jnp.int32, (NPAD, G), 1)
        oh = (bt_ref[...] == cols).astype(f32)
        ones = jnp.ones((NPAD, 1), f32)
        cnt = jnp.maximum(
            lax.dot_general(oh, ones, (((0,), (0,)), ((), ())),
                            preferred_element_type=f32), 1.0)
        g = lax.dot_general(oh, h_ref[...], (((0,), (0,)), ((), ())),
                            preferred_element_type=f32)
        g = g / cnt
        y = jnp.maximum(jnp.dot(g, w1_ref[...], preferred_element_type=f32)
                        + b1_ref[...], 0.0)
        out_ref[...] = (jnp.dot(y, w2_ref[...], preferred_element_type=f32)
                        + b2_ref[...])

    sds = jax.ShapeDtypeStruct
    init_call = pl.pallas_call(
        _init_body,
        out_shape=(sds((NPAD, H), f32), sds((NPAD, 2 * H), f32),
                   sds((NPAD, 2 * H), f32)))

    RB = 4096
    assert EW % RB == 0
    c_call = pl.pallas_call(
        _c_body,
        grid=(EW // RB,),
        in_specs=[pl.BlockSpec((RB, DE), lambda i: (i, 0)),
                  pl.BlockSpec((DE, 2 * H), lambda i: (0, 0)),
                  pl.BlockSpec((1, 2 * H), lambda i: (0, 0))],
        out_specs=pl.BlockSpec((RB, 2 * H), lambda i: (i, 0)),
        out_shape=sds((EW, 2 * H), f32))

    node_ab_call = pl.pallas_call(
        _node_ab_body,
        out_shape=(sds((NPAD, H), f32), sds((NPAD, 2 * H), f32),
                   sds((NPAD, 2 * H), f32)))
    node_last_call = pl.pallas_call(_node_last_body,
                                    out_shape=sds((NPAD, H), f32))
    pool_call = pl.pallas_call(_pool_body, out_shape=sds((G, 1), f32))

    edge_call = _make_edge_kernel(NPAD, nb, H)

    h, A, B = init_call(xp, emb, Wd[0], Wsr[0])
    for l in range(L):
        Cl = c_call(eap, We[l], bcat[l].reshape(1, -1))
        parts = edge_call(A, B, Cl, dstp, srcp)
        if l < L - 1:
            h, A, B = node_ab_call(parts, h, gamma[l].reshape(1, -1),
                                   beta[l].reshape(1, -1), Wd[l + 1],
                                   Wsr[l + 1])
        else:
            h = node_last_call(parts, h, gamma[l].reshape(1, -1),
                               beta[l].reshape(1, -1))
    out = pool_call(h, batchp, W1, b1.reshape(1, -1), W2, b2.reshape(1, 1))
    return out.reshape(-1)


# speedup vs baseline: 1.1226x; 1.0148x over previous
import functools

import jax
import jax.numpy as jnp
from jax import lax
from jax.experimental import pallas as pl
from jax.experimental.pallas import tpu as pltpu
from jax.experimental.pallas import tpu_sc as plsc

NC = 2
NS = 16
NW = NC * NS
LANES = 16
K = 32
CH = 32


def _sigmoid(x):
    return 1.0 / (1.0 + jnp.exp(-x))


def _softplus(x):
    u = jnp.exp(-jnp.abs(x))
    t = u / (u + 2.0)
    w = t * t
    p = 1.0 + w * (1.0 / 3.0 + w * (0.2 + w * (1.0 / 7.0 + w * (1.0 / 9.0))))
    return jnp.maximum(x, 0.0) + 2.0 * t * p


def _make_edge_kernel(NPAD, nb, HH):
    EPT = nb * K
    RW = HH + LANES
    RPT = NPAD // NS
    W2H = 2 * HH
    NCHUNK = nb // CH
    mesh = plsc.VectorSubcoreMesh(core_axis_name="c", subcore_axis_name="s")

    @functools.partial(
        pl.kernel,
        out_type=jax.ShapeDtypeStruct((NC, NPAD, RW), jnp.float32),
        mesh=mesh,
        compiler_params=pltpu.CompilerParams(use_tc_tiling_on_sc=False,
                                             needs_layout_passes=False),
        scratch_types=[
            pltpu.VMEM((K, W2H), jnp.bfloat16),
            pltpu.VMEM((K, W2H), jnp.bfloat16),
            pltpu.VMEM((K, W2H), jnp.bfloat16),
            pltpu.VMEM((K, W2H), jnp.bfloat16),
            pltpu.VMEM((K, W2H), jnp.bfloat16),
            pltpu.VMEM((K, W2H), jnp.bfloat16),
            pltpu.VMEM((K, RW), jnp.float32),
            pltpu.VMEM((K, RW), jnp.float32),
            pltpu.VMEM((CH, K), jnp.int32),
            pltpu.VMEM((CH, K), jnp.int32),
            pltpu.VMEM((K,), jnp.int32),
            pltpu.VMEM((K,), jnp.int32),
            pltpu.VMEM_SHARED((NPAD, RW), jnp.float32),
            pltpu.SemaphoreType.DMA,
            pltpu.SemaphoreType.DMA,
            pltpu.SemaphoreType.DMA,
        ],
    )
    def edge_kernel(a_hbm, b_hbm, c_hbm, dst_hbm, src_hbm, out_hbm,
                    a0, a1, b0, b1, c0, c1, m0, m1, dch, sch, dsc0, dsc1,
                    shared, sem_g0, sem_g1, sem_s):
        cid = lax.axis_index("c")
        sid = lax.axis_index("s")
        w = cid * NS + sid
        abufs = (a0, a1)
        bbufs = (b0, b1)
        cbufs = (c0, c1)
        mbufs = (m0, m1)
        dscs = (dsc0, dsc1)
        gsems = (sem_g0, sem_g1)

        zvec = jnp.zeros((LANES,), jnp.float32)

        @pl.loop(0, K)
        def _zero_m(j):
            for qq in range(RW // LANES):
                m0[j, pl.ds(qq * LANES, LANES)] = zvec

        base = sid * RPT
        nfull = RPT // K
        rem = RPT - nfull * K

        @pl.loop(0, nfull)
        def _zcp(z):
            pltpu.sync_copy(m0, shared.at[pl.ds(base + z * K, K)])

        if rem:
            pltpu.sync_copy(m0.at[pl.ds(0, rem)],
                            shared.at[pl.ds(base + nfull * K, rem)])

        ji = lax.iota(jnp.int32, LANES)
        unit = jnp.where(ji == 0, 1.0, 0.0).astype(jnp.float32)

        @pl.loop(0, K)
        def _pad_m(j):
            m0[j, pl.ds(HH, LANES)] = unit
            m1[j, pl.ds(HH, LANES)] = unit

        plsc.subcore_barrier()

        @pl.loop(0, NCHUNK)
        def _chunk(c):
            pltpu.sync_copy(dst_hbm.at[w, c], dch)
            pltpu.sync_copy(src_hbm.at[w, c], sch)
            base_e = w * EPT + c * (CH * K)
            pltpu.async_copy(a_hbm.at[dch.at[0]], a0, sem_g0)
            pltpu.async_copy(b_hbm.at[sch.at[0]], b0, sem_g0)
            pltpu.async_copy(c_hbm.at[pl.ds(base_e, K)], c0, sem_g0)

            @pl.loop(0, CH, step=2)
            def _blk(J):
                for j in range(2):
                    p = j
                    b = J + j
                    A_, B_, C_, M_ = abufs[p], bbufs[p], cbufs[p], mbufs[p]
                    An, Bn, Cn = abufs[1 - p], bbufs[1 - p], cbufs[1 - p]
                    Mp = mbufs[1 - p]
                    pltpu.make_async_copy(
                        a_hbm.at[dch.at[b]], A_, gsems[p]).wait()
                    pltpu.make_async_copy(
                        b_hbm.at[sch.at[b]], B_, gsems[p]).wait()
                    pltpu.make_async_copy(
                        c_hbm.at[pl.ds(base_e + b * K, K)], C_,
                        gsems[p]).wait()

                    @pl.when(b + 1 < CH)
                    def _pf():
                        pltpu.async_copy(
                            a_hbm.at[dch.at[b + 1]], An, gsems[1 - p])
                        pltpu.async_copy(
                            b_hbm.at[sch.at[b + 1]], Bn, gsems[1 - p])
                        pltpu.async_copy(
                            c_hbm.at[pl.ds(base_e + (b + 1) * K, K)],
                            Cn, gsems[1 - p])

                    @pl.loop(0, K)
                    def _edge(jj):
                        for g in range(HH // LANES):
                            lo = 2 * g * LANES
                            afv, asv = plsc.unpack(
                                A_[jj, pl.ds(lo, 2 * LANES)],
                                format=plsc.PackFormat.INTERLEAVED)
                            bfv, bsv = plsc.unpack(
                                B_[jj, pl.ds(lo, 2 * LANES)],
                                format=plsc.PackFormat.INTERLEAVED)
                            cfv, csv = plsc.unpack(
                                C_[jj, pl.ds(lo, 2 * LANES)],
                                format=plsc.PackFormat.INTERLEAVED)
                            zf = afv + bfv + cfv
                            zs = asv + bsv + csv
                            M_[jj, pl.ds(g * LANES, LANES)] = (
                                _sigmoid(zf) * _softplus(zs))

                    for q2 in range(K // LANES):
                        dscs[p][pl.ds(q2 * LANES, LANES)] = (
                            dch[b, pl.ds(q2 * LANES, LANES)])

                    @pl.when(b > 0)
                    def _ws():
                        pltpu.make_async_copy(
                            Mp, shared.at[dscs[1 - p]], sem_s).wait()

                    pltpu.async_copy(M_, shared.at[dscs[p]], sem_s,
                                     add=True)

            pltpu.make_async_copy(m1, shared.at[dsc1], sem_s).wait()

        plsc.subcore_barrier()
        pltpu.sync_copy(shared.at[pl.ds(base, RPT)],
                        out_hbm.at[cid, pl.ds(base, RPT)])

    return edge_kernel


def kernel(x, edge_index, edge_attr, batch, emb, Wf, bf, Ws, bs,
           gamma, beta, W1, b1, W2, b2):
    N = x.shape[0]
    E = edge_index.shape[1]
    H = emb.shape[1]
    DE = edge_attr.shape[1]
    L = Wf.shape[0]
    FC = W1.shape[1]
    G = 64
    f32 = jnp.float32

    NPAD = ((N + LANES + 127) // 128) * 128
    nb = -(-E // (NW * K))
    nb = ((nb + CH - 1) // CH) * CH
    EW = NW * nb * K
    RW = H + LANES

    idt = edge_index.dtype
    dstp = jnp.concatenate(
        [edge_index[1], jnp.full((EW - E,), N, idt)]).reshape(
            NW, nb // CH, CH, K)
    srcp = jnp.concatenate(
        [edge_index[0], jnp.full((EW - E,), N, idt)]).reshape(
            NW, nb // CH, CH, K)
    eap = jnp.concatenate(
        [edge_attr, jnp.zeros((EW - E, DE), f32)], axis=0)
    xp = jnp.concatenate([x, jnp.zeros((NPAD - N, 1), x.dtype)], axis=0)
    batchp = jnp.concatenate(
        [batch, jnp.full((NPAD - N,), G, batch.dtype)]).reshape(NPAD, 1)
    Wd = jnp.concatenate([Wf[:, :H, :], Ws[:, :H, :]], axis=2)
    Wsr = jnp.concatenate([Wf[:, H:2 * H, :], Ws[:, H:2 * H, :]], axis=2)
    We = jnp.concatenate([Wf[:, 2 * H:, :], Ws[:, 2 * H:, :]], axis=2)
    bcat = jnp.concatenate([bf, bs], axis=1)
    perm = []
    for qg in range(H // LANES):
        for i in range(LANES):
            perm.append(qg * LANES + i)
            perm.append(H + qg * LANES + i)
    perm = jnp.array(perm, jnp.int32)
    Wd = Wd[:, :, perm]
    Wsr = Wsr[:, :, perm]
    We = We[:, :, perm]
    bcat = bcat[:, perm]

    def _init_body(x_ref, emb_ref, wd_ref, ws_ref, h_ref, a_ref, b_ref):
        z = jnp.clip(x_ref[...], 0, 119)
        cols = lax.broadcasted_iota(jnp.int32, (NPAD, 120), 1)
        rows = lax.broadcasted_iota(jnp.int32, (NPAD, 1), 0)
        oh = jnp.where((z == cols) & (rows < N), 1.0, 0.0).astype(f32)
        h = jnp.dot(oh, emb_ref[...], preferred_element_type=f32)
        h_ref[...] = h
        a_ref[...] = jnp.dot(
            h, wd_ref[...], preferred_element_type=f32).astype(jnp.bfloat16)
        b_ref[...] = jnp.dot(
            h, ws_ref[...], preferred_element_type=f32).astype(jnp.bfloat16)

    def _c_body(ea_ref, w_ref, bv_ref, c_ref):
        c_ref[...] = (jnp.dot(ea_ref[...], w_ref[...],
                              preferred_element_type=f32)
                      + bv_ref[...]).astype(jnp.bfloat16)

    def _node_core(parts_ref, h_ref, g_ref, be_ref):
        sm = parts_ref[0] + parts_ref[1]
        deg = jnp.maximum(sm[:, H:H + 1], 1.0)
        aggr = sm[:, :H] / deg
        rows = lax.broadcasted_iota(jnp.int32, (NPAD, 1), 0)
        msk = (rows < N).astype(f32)
        aggr = aggr * msk
        mu = jnp.sum(aggr, axis=0, keepdims=True) / N
        d = (aggr - mu) * msk
        var = jnp.sum(d * d, axis=0, keepdims=True) / N
        nrm = d * lax.rsqrt(var + 1e-5) * g_ref[...] + be_ref[...]
        return jnp.maximum(h_ref[...] + nrm, 0.0) * msk

    def _node_ab_body(parts_ref, h_ref, g_ref, be_ref, wd_ref, ws_ref,
                      h2_ref, a_ref, b_ref):
        h2 = _node_core(parts_ref, h_ref, g_ref, be_ref)
        h2_ref[...] = h2
        a_ref[...] = jnp.dot(
            h2, wd_ref[...], preferred_element_type=f32).astype(jnp.bfloat16)
        b_ref[...] = jnp.dot(
            h2, ws_ref[...], preferred_element_type=f32).astype(jnp.bfloat16)

    def _node_last_body(parts_ref, h_ref, g_ref, be_ref, h2_ref):
        h2_ref[...] = _node_core(parts_ref, h_ref, g_ref, be_ref)

    def _pool_body(h_ref, bt_ref, w1_ref, b1_ref, w2_ref, b2_ref, out_ref):
        cols = lax.broadcasted_iota(jnp.int32, (NPAD, G), 1)
        oh = (bt_ref[...] == cols).astype(f32)
        ones = jnp.ones((NPAD, 1), f32)
        cnt = jnp.maximum(
            lax.dot_general(oh, ones, (((0,), (0,)), ((), ())),
                            preferred_element_type=f32), 1.0)
        g = lax.dot_general(oh, h_ref[...], (((0,), (0,)), ((), ())),
                            preferred_element_type=f32)
        g = g / cnt
        y = jnp.maximum(jnp.dot(g, w1_ref[...], preferred_element_type=f32)
                        + b1_ref[...], 0.0)
        out_ref[...] = (jnp.dot(y, w2_ref[...], preferred_element_type=f32)
                        + b2_ref[...])

    sds = jax.ShapeDtypeStruct
    bf16 = jnp.bfloat16
    init_call = pl.pallas_call(
        _init_body,
        out_shape=(sds((NPAD, H), f32), sds((NPAD, 2 * H), bf16),
                   sds((NPAD, 2 * H), bf16)))

    RB = 4096
    assert EW % RB == 0
    c_call = pl.pallas_call(
        _c_body,
        grid=(EW // RB,),
        in_specs=[pl.BlockSpec((RB, DE), lambda i: (i, 0)),
                  pl.BlockSpec((DE, 2 * H), lambda i: (0, 0)),
                  pl.BlockSpec((1, 2 * H), lambda i: (0, 0))],
        out_specs=pl.BlockSpec((RB, 2 * H), lambda i: (i, 0)),
        out_shape=sds((EW, 2 * H), bf16))

    node_ab_call = pl.pallas_call(
        _node_ab_body,
        out_shape=(sds((NPAD, H), f32), sds((NPAD, 2 * H), bf16),
                   sds((NPAD, 2 * H), bf16)))
    node_last_call = pl.pallas_call(_node_last_body,
                                    out_shape=sds((NPAD, H), f32))
    pool_call = pl.pallas_call(_pool_body, out_shape=sds((G, 1), f32))

    edge_call = _make_edge_kernel(NPAD, nb, H)

    h, A, B = init_call(xp, emb, Wd[0], Wsr[0])
    for l in range(L):
        Cl = c_call(eap, We[l], bcat[l].reshape(1, -1))
        parts = edge_call(A, B, Cl, dstp, srcp)
        if l < L - 1:
            h, A, B = node_ab_call(parts, h, gamma[l].reshape(1, -1),
                                   beta[l].reshape(1, -1), Wd[l + 1],
                                   Wsr[l + 1])
        else:
            h = node_last_call(parts, h, gamma[l].reshape(1, -1),
                               beta[l].reshape(1, -1))
    out = pool_call(h, batchp, W1, b1.reshape(1, -1), W2, b2.reshape(1, 1))
    return out.reshape(-1)


# speedup vs baseline: 1.1731x; 1.0450x over previous
import functools

import jax
import jax.numpy as jnp
from jax import lax
from jax.experimental import pallas as pl
from jax.experimental.pallas import tpu as pltpu
from jax.experimental.pallas import tpu_sc as plsc

NC = 2
NS = 16
NW = NC * NS
LANES = 16
K = 32
CH = 32


def _sigmoid(x):
    return 1.0 / (1.0 + jnp.exp(-x))


def _softplus(x):
    u = jnp.exp(-jnp.abs(x))
    t = u / (u + 2.0)
    w = t * t
    p = 1.0 + w * (1.0 / 3.0 + w * (0.2 + w * (1.0 / 7.0 + w * (1.0 / 9.0))))
    return jnp.maximum(x, 0.0) + 2.0 * t * p


def _make_edge_kernel(NPAD, nb, HH):
    EPT = nb * K
    RW = HH + LANES
    RPT = NPAD // NS
    W2H = 2 * HH
    NCHUNK = nb // CH
    mesh = plsc.VectorSubcoreMesh(core_axis_name="c", subcore_axis_name="s")

    @functools.partial(
        pl.kernel,
        out_type=jax.ShapeDtypeStruct((NC, NPAD, RW), jnp.float32),
        mesh=mesh,
        compiler_params=pltpu.CompilerParams(use_tc_tiling_on_sc=False,
                                             needs_layout_passes=False),
        scratch_types=[
            pltpu.VMEM((K, W2H), jnp.bfloat16),
            pltpu.VMEM((K, W2H), jnp.bfloat16),
            pltpu.VMEM((K, W2H), jnp.bfloat16),
            pltpu.VMEM((K, W2H), jnp.bfloat16),
            pltpu.VMEM((K, W2H), jnp.bfloat16),
            pltpu.VMEM((K, W2H), jnp.bfloat16),
            pltpu.VMEM((K, RW), jnp.float32),
            pltpu.VMEM((K, RW), jnp.float32),
            pltpu.VMEM((CH, K), jnp.int32),
            pltpu.VMEM((CH, K), jnp.int32),
            pltpu.VMEM((K,), jnp.int32),
            pltpu.VMEM((K,), jnp.int32),
            pltpu.VMEM_SHARED((NPAD, RW), jnp.float32),
            pltpu.SemaphoreType.DMA,
            pltpu.SemaphoreType.DMA,
            pltpu.SemaphoreType.DMA,
        ],
    )
    def edge_kernel(a_hbm, b_hbm, c_hbm, dst_hbm, src_hbm, out_hbm,
                    a0, a1, b0, b1, c0, c1, m0, m1, dch, sch, dsc0, dsc1,
                    shared, sem_g0, sem_g1, sem_s):
        cid = lax.axis_index("c")
        sid = lax.axis_index("s")
        w = cid * NS + sid
        abufs = (a0, a1)
        bbufs = (b0, b1)
        cbufs = (c0, c1)
        mbufs = (m0, m1)
        dscs = (dsc0, dsc1)
        gsems = (sem_g0, sem_g1)

        zvec = jnp.zeros((LANES,), jnp.float32)

        @pl.loop(0, K)
        def _zero_m(j):
            for qq in range(RW // LANES):
                m0[j, pl.ds(qq * LANES, LANES)] = zvec

        base = sid * RPT
        nfull = RPT // K
        rem = RPT - nfull * K

        @pl.loop(0, nfull)
        def _zcp(z):
            pltpu.sync_copy(m0, shared.at[pl.ds(base + z * K, K)])

        if rem:
            pltpu.sync_copy(m0.at[pl.ds(0, rem)],
                            shared.at[pl.ds(base + nfull * K, rem)])

        ji = lax.iota(jnp.int32, LANES)
        unit = jnp.where(ji == 0, 1.0, 0.0).astype(jnp.float32)

        @pl.loop(0, K)
        def _pad_m(j):
            m0[j, pl.ds(HH, LANES)] = unit
            m1[j, pl.ds(HH, LANES)] = unit

        plsc.subcore_barrier()

        @pl.loop(0, NCHUNK)
        def _chunk(c):
            pltpu.sync_copy(dst_hbm.at[w, c], dch)
            pltpu.sync_copy(src_hbm.at[w, c], sch)
            base_e = w * EPT + c * (CH * K)
            pltpu.async_copy(a_hbm.at[dch.at[0]], a0, sem_g0)
            pltpu.async_copy(b_hbm.at[sch.at[0]], b0, sem_g0)
            pltpu.async_copy(c_hbm.at[pl.ds(base_e, K)], c0, sem_g0)

            @pl.loop(0, CH, step=2)
            def _blk(J):
                for j in range(2):
                    p = j
                    b = J + j
                    A_, B_, C_, M_ = abufs[p], bbufs[p], cbufs[p], mbufs[p]
                    An, Bn, Cn = abufs[1 - p], bbufs[1 - p], cbufs[1 - p]
                    Mp = mbufs[1 - p]
                    pltpu.make_async_copy(
                        a_hbm.at[dch.at[b]], A_, gsems[p]).wait()
                    pltpu.make_async_copy(
                        b_hbm.at[sch.at[b]], B_, gsems[p]).wait()
                    pltpu.make_async_copy(
                        c_hbm.at[pl.ds(base_e + b * K, K)], C_,
                        gsems[p]).wait()

                    @pl.when(b + 1 < CH)
                    def _pf():
                        pltpu.async_copy(
                            a_hbm.at[dch.at[b + 1]], An, gsems[1 - p])
                        pltpu.async_copy(
                            b_hbm.at[sch.at[b + 1]], Bn, gsems[1 - p])
                        pltpu.async_copy(
                            c_hbm.at[pl.ds(base_e + (b + 1) * K, K)],
                            Cn, gsems[1 - p])

                    @plsc.parallel_loop(0, K, unroll=4)
                    def _edge(jj):
                        for g in range(HH // LANES):
                            lo = 2 * g * LANES
                            afv, asv = plsc.unpack(
                                A_[jj, pl.ds(lo, 2 * LANES)],
                                format=plsc.PackFormat.INTERLEAVED)
                            bfv, bsv = plsc.unpack(
                                B_[jj, pl.ds(lo, 2 * LANES)],
                                format=plsc.PackFormat.INTERLEAVED)
                            cfv, csv = plsc.unpack(
                                C_[jj, pl.ds(lo, 2 * LANES)],
                                format=plsc.PackFormat.INTERLEAVED)
                            zf = afv + bfv + cfv
                            zs = asv + bsv + csv
                            M_[jj, pl.ds(g * LANES, LANES)] = (
                                _sigmoid(zf) * _softplus(zs))

                    for q2 in range(K // LANES):
                        dscs[p][pl.ds(q2 * LANES, LANES)] = (
                            dch[b, pl.ds(q2 * LANES, LANES)])

                    @pl.when(b > 0)
                    def _ws():
                        pltpu.make_async_copy(
                            Mp, shared.at[dscs[1 - p]], sem_s).wait()

                    pltpu.async_copy(M_, shared.at[dscs[p]], sem_s,
                                     add=True)

            pltpu.make_async_copy(m1, shared.at[dsc1], sem_s).wait()

        plsc.subcore_barrier()
        pltpu.sync_copy(shared.at[pl.ds(base, RPT)],
                        out_hbm.at[cid, pl.ds(base, RPT)])

    return edge_kernel


def kernel(x, edge_index, edge_attr, batch, emb, Wf, bf, Ws, bs,
           gamma, beta, W1, b1, W2, b2):
    N = x.shape[0]
    E = edge_index.shape[1]
    H = emb.shape[1]
    DE = edge_attr.shape[1]
    L = Wf.shape[0]
    FC = W1.shape[1]
    G = 64
    f32 = jnp.float32

    NPAD = ((N + LANES + 127) // 128) * 128
    nb = -(-E // (NW * K))
    nb = ((nb + CH - 1) // CH) * CH
    EW = NW * nb * K
    RW = H + LANES

    idt = edge_index.dtype
    dstp = jnp.concatenate(
        [edge_index[1], jnp.full((EW - E,), N, idt)]).reshape(
            NW, nb // CH, CH, K)
    srcp = jnp.concatenate(
        [edge_index[0], jnp.full((EW - E,), N, idt)]).reshape(
            NW, nb // CH, CH, K)
    eap = jnp.concatenate(
        [edge_attr, jnp.zeros((EW - E, DE), f32)], axis=0)
    xp = jnp.concatenate([x, jnp.zeros((NPAD - N, 1), x.dtype)], axis=0)
    batchp = jnp.concatenate(
        [batch, jnp.full((NPAD - N,), G, batch.dtype)]).reshape(NPAD, 1)
    Wd = jnp.concatenate([Wf[:, :H, :], Ws[:, :H, :]], axis=2)
    Wsr = jnp.concatenate([Wf[:, H:2 * H, :], Ws[:, H:2 * H, :]], axis=2)
    We = jnp.concatenate([Wf[:, 2 * H:, :], Ws[:, 2 * H:, :]], axis=2)
    bcat = jnp.concatenate([bf, bs], axis=1)
    perm = []
    for qg in range(H // LANES):
        for i in range(LANES):
            perm.append(qg * LANES + i)
            perm.append(H + qg * LANES + i)
    perm = jnp.array(perm, jnp.int32)
    Wd = Wd[:, :, perm]
    Wsr = Wsr[:, :, perm]
    We = We[:, :, perm]
    bcat = bcat[:, perm]

    def _init_body(x_ref, emb_ref, wd_ref, ws_ref, h_ref, a_ref, b_ref):
        z = jnp.clip(x_ref[...], 0, 119)
        cols = lax.broadcasted_iota(jnp.int32, (NPAD, 120), 1)
        rows = lax.broadcasted_iota(jnp.int32, (NPAD, 1), 0)
        oh = jnp.where((z == cols) & (rows < N), 1.0, 0.0).astype(f32)
        h = jnp.dot(oh, emb_ref[...], preferred_element_type=f32)
        h_ref[...] = h
        a_ref[...] = jnp.dot(
            h, wd_ref[...], preferred_element_type=f32).astype(jnp.bfloat16)
        b_ref[...] = jnp.dot(
            h, ws_ref[...], preferred_element_type=f32).astype(jnp.bfloat16)

    def _c_body(ea_ref, w_ref, bv_ref, c_ref):
        c_ref[...] = (jnp.dot(ea_ref[...], w_ref[...],
                              preferred_element_type=f32)
                      + bv_ref[...]).astype(jnp.bfloat16)

    def _node_core(parts_ref, h_ref, g_ref, be_ref):
        sm = parts_ref[0] + parts_ref[1]
        deg = jnp.maximum(sm[:, H:H + 1], 1.0)
        aggr = sm[:, :H] / deg
        rows = lax.broadcasted_iota(jnp.int32, (NPAD, 1), 0)
        msk = (rows < N).astype(f32)
        aggr = aggr * msk
        mu = jnp.sum(aggr, axis=0, keepdims=True) / N
        d = (aggr - mu) * msk
        var = jnp.sum(d * d, axis=0, keepdims=True) / N
        nrm = d * lax.rsqrt(var + 1e-5) * g_ref[...] + be_ref[...]
        return jnp.maximum(h_ref[...] + nrm, 0.0) * msk

    def _node_ab_body(parts_ref, h_ref, g_ref, be_ref, wd_ref, ws_ref,
                      h2_ref, a_ref, b_ref):
        h2 = _node_core(parts_ref, h_ref, g_ref, be_ref)
        h2_ref[...] = h2
        a_ref[...] = jnp.dot(
            h2, wd_ref[...], preferred_element_type=f32).astype(jnp.bfloat16)
        b_ref[...] = jnp.dot(
            h2, ws_ref[...], preferred_element_type=f32).astype(jnp.bfloat16)

    def _node_last_body(parts_ref, h_ref, g_ref, be_ref, h2_ref):
        h2_ref[...] = _node_core(parts_ref, h_ref, g_ref, be_ref)

    def _pool_body(h_ref, bt_ref, w1_ref, b1_ref, w2_ref, b2_ref, out_ref):
        cols = lax.broadcasted_iota(jnp.int32, (NPAD, G), 1)
        oh = (bt_ref[...] == cols).astype(f32)
        ones = jnp.ones((NPAD, 1), f32)
        cnt = jnp.maximum(
            lax.dot_general(oh, ones, (((0,), (0,)), ((), ())),
                            preferred_element_type=f32), 1.0)
        g = lax.dot_general(oh, h_ref[...], (((0,), (0,)), ((), ())),
                            preferred_element_type=f32)
        g = g / cnt
        y = jnp.maximum(jnp.dot(g, w1_ref[...], preferred_element_type=f32)
                        + b1_ref[...], 0.0)
        out_ref[...] = (jnp.dot(y, w2_ref[...], preferred_element_type=f32)
                        + b2_ref[...])

    sds = jax.ShapeDtypeStruct
    bf16 = jnp.bfloat16
    init_call = pl.pallas_call(
        _init_body,
        out_shape=(sds((NPAD, H), f32), sds((NPAD, 2 * H), bf16),
                   sds((NPAD, 2 * H), bf16)))

    RB = 4096
    assert EW % RB == 0
    c_call = pl.pallas_call(
        _c_body,
        grid=(EW // RB,),
        in_specs=[pl.BlockSpec((RB, DE), lambda i: (i, 0)),
                  pl.BlockSpec((DE, 2 * H), lambda i: (0, 0)),
                  pl.BlockSpec((1, 2 * H), lambda i: (0, 0))],
        out_specs=pl.BlockSpec((RB, 2 * H), lambda i: (i, 0)),
        out_shape=sds((EW, 2 * H), bf16))

    node_ab_call = pl.pallas_call(
        _node_ab_body,
        out_shape=(sds((NPAD, H), f32), sds((NPAD, 2 * H), bf16),
                   sds((NPAD, 2 * H), bf16)))
    node_last_call = pl.pallas_call(_node_last_body,
                                    out_shape=sds((NPAD, H), f32))
    pool_call = pl.pallas_call(_pool_body, out_shape=sds((G, 1), f32))

    edge_call = _make_edge_kernel(NPAD, nb, H)

    h, A, B = init_call(xp, emb, Wd[0], Wsr[0])
    for l in range(L):
        Cl = c_call(eap, We[l], bcat[l].reshape(1, -1))
        parts = edge_call(A, B, Cl, dstp, srcp)
        if l < L - 1:
            h, A, B = node_ab_call(parts, h, gamma[l].reshape(1, -1),
                                   beta[l].reshape(1, -1), Wd[l + 1],
                                   Wsr[l + 1])
        else:
            h = node_last_call(parts, h, gamma[l].reshape(1, -1),
                               beta[l].reshape(1, -1))
    out = pool_call(h, batchp, W1, b1.reshape(1, -1), W2, b2.reshape(1, 1))
    return out.reshape(-1)
